# baseline (speedup 1.0000x reference)
import functools

import jax
import jax.numpy as jnp
from jax import lax
from jax.experimental import pallas as pl
from jax.experimental.pallas import tpu as pltpu
from jax.experimental.pallas import tpu_sc as plsc

B, H, W = 8, 512, 512
HW = H * W

_LANES = 128
_ROWS = B * HW // _LANES
_NW = 32
_RPW = _ROWS // _NW
_CH = 16
_NCH = _RPW // _CH


def _lane_cumsum_excl(x):
    total = x
    sh = 1
    while sh < W:
        z = jnp.zeros((1, sh), jnp.float32)
        total = total + jnp.concatenate([z, total[:, :-sh]], axis=1)
        sh *= 2
    return total - x


def _dense_body(tri_ref, mask_ref, omega_ref, idxm_ref, kk_ref, dest_ref,
                vii_ref, vjj_ref):
    b = pl.program_id(0)
    m = mask_ref[0]
    zrow = jnp.zeros((1, W), jnp.float32)
    zcol = jnp.zeros((H, 1), jnp.float32)
    m_down = jnp.concatenate([m[1:, :], zrow], axis=0)
    m_up = jnp.concatenate([zrow, m[:-1, :]], axis=0)
    m_right = jnp.concatenate([m[:, 1:], zcol], axis=1)
    m_left = jnp.concatenate([zcol, m[:, :-1]], axis=1)
    o0 = m * m_down
    o1 = m * m_up
    o2 = m * m_right
    o3 = m * m_left
    omega_ref[0, 0] = o0
    omega_ref[0, 1] = o1
    omega_ref[0, 2] = o2
    omega_ref[0, 3] = o3

    tri = tri_ref[...]

    c1 = jnp.dot(tri, m.astype(jnp.bfloat16),
                 preferred_element_type=jnp.float32)
    e1 = _lane_cumsum_excl(c1[H - 1:H, :])
    ranks = c1 + e1
    idxm = ranks * m
    idxm_ref[0] = idxm

    c2 = jnp.dot(tri, o2.astype(jnp.bfloat16),
                 preferred_element_type=jnp.float32)
    s2 = c2[H - 1:H, :]
    e2 = _lane_cumsum_excl(s2)
    r2 = c2 + e2
    counts = e2[:, W - 1:W] + s2[:, W - 1:W]

    vii_ref[0] = (idxm * o2).reshape(H * W // _LANES, _LANES)
    vjj_ref[0] = (jnp.concatenate([idxm[:, 1:], zcol], axis=1)
                  * o2).reshape(H * W // _LANES, _LANES)

    r2i = r2.astype(jnp.int32)
    counts_i = counts.astype(jnp.int32)
    hh = lax.broadcasted_iota(jnp.int32, (H, W), 0)
    ww = lax.broadcasted_iota(jnp.int32, (H, W), 1)
    j = ww * H + hh
    dest = jnp.where(o2 != 0, r2i - 1, counts_i + j - r2i)
    dest_ref[0] = (dest + lax.rem(b, 2) * HW).reshape(H * W // _LANES, _LANES)

    pos = lax.broadcasted_iota(jnp.int32, (1, 2 * HW), 1)
    sign = jnp.where(pos < HW, 1.0, -1.0)
    kk_ref[0] = jnp.where((pos & (HW - 1)) < counts_i, sign, 0.0)


@jax.jit
def _dense_pass(tri, mask):
    return pl.pallas_call(
        _dense_body,
        grid=(B,),
        in_specs=[
            pl.BlockSpec((H, H), lambda b: (0, 0)),
            pl.BlockSpec((1, H, W), lambda b: (b, 0, 0)),
        ],
        out_specs=[
            pl.BlockSpec((1, 4, H, W), lambda b: (b, 0, 0, 0)),
            pl.BlockSpec((1, H, W), lambda b: (b, 0, 0)),
            pl.BlockSpec((1, 1, 2 * HW), lambda b: (b, 0, 0)),
            pl.BlockSpec((1, HW // _LANES, _LANES), lambda b: (b, 0, 0)),
            pl.BlockSpec((1, HW // _LANES, _LANES), lambda b: (b, 0, 0)),
            pl.BlockSpec((1, HW // _LANES, _LANES), lambda b: (b, 0, 0)),
        ],
        out_shape=[
            jax.ShapeDtypeStruct((B, 4, H, W), jnp.float32),
            jax.ShapeDtypeStruct((B, H, W), jnp.float32),
            jax.ShapeDtypeStruct((B, 1, 2 * HW), jnp.float32),
            jax.ShapeDtypeStruct((B, HW // _LANES, _LANES), jnp.int32),
            jax.ShapeDtypeStruct((B, HW // _LANES, _LANES), jnp.float32),
            jax.ShapeDtypeStruct((B, HW // _LANES, _LANES), jnp.float32),
        ],
        compiler_params=pltpu.CompilerParams(
            dimension_semantics=("arbitrary",)),
    )(tri, mask)


_PAIR = 2 * HW
_RPT = 2048 * 2 // 16
_NCH2 = _RPT // _CH
_WR = _PAIR // 16


def _scatter_body(dest_hbm, vii_hbm, vjj_hbm, ii_out, jj_out,
                  sp_ii, sp_jj, idx_v, a_v, b_v, sem_a, sem_b):
    c = lax.axis_index("c")
    s = lax.axis_index("s")

    def one_round(r):
        b = 4 * c + 2 * r + s // 8
        lrow0 = (s % 8) * _RPT

        def chunk(i, carry):
            rr = lrow0 + i * _CH
            pltpu.sync_copy(dest_hbm.at[b, pl.ds(rr, _CH)], idx_v)
            pltpu.sync_copy(vii_hbm.at[b, pl.ds(rr, _CH)], a_v)
            pltpu.sync_copy(vjj_hbm.at[b, pl.ds(rr, _CH)], b_v)

            def fire(j, carry2):
                pltpu.async_copy(a_v.at[j], sp_ii.at[idx_v.at[j]], sem_a)
                pltpu.async_copy(b_v.at[j], sp_jj.at[idx_v.at[j]], sem_b)
                return carry2

            lax.fori_loop(0, _CH, fire, 0)

            def drain(j, carry2):
                pltpu.make_async_copy(a_v.at[j], sp_ii.at[idx_v.at[j]],
                                      sem_a).wait()
                pltpu.make_async_copy(b_v.at[j], sp_jj.at[idx_v.at[j]],
                                      sem_b).wait()
                return carry2

            lax.fori_loop(0, _CH, drain, 0)
            return carry

        lax.fori_loop(0, _NCH2, chunk, 0)
        plsc.subcore_barrier()
        off = (s % 8) * _WR
        src_ii = sp_ii.at[pl.ds(s * _WR, _WR)]
        src_jj = sp_jj.at[pl.ds(s * _WR, _WR)]
        pltpu.sync_copy(src_ii, ii_out.at[b, pl.ds(off, _WR)])
        pltpu.sync_copy(src_ii, ii_out.at[b, pl.ds(HW + off, _WR)])
        pltpu.sync_copy(src_ii, jj_out.at[b, pl.ds(HW + off, _WR)])
        pltpu.sync_copy(src_jj, jj_out.at[b, pl.ds(off, _WR)])
        plsc.subcore_barrier()

    one_round(0)
    one_round(1)


@jax.jit
def _scatter_pass(dest, vii, vjj):
    return pl.kernel(
        _scatter_body,
        out_type=[
            jax.ShapeDtypeStruct((B, 2 * HW), jnp.float32),
            jax.ShapeDtypeStruct((B, 2 * HW), jnp.float32),
        ],
        mesh=plsc.VectorSubcoreMesh(core_axis_name="c", subcore_axis_name="s"),
        scratch_types=[
            pltpu.VMEM_SHARED((_PAIR,), jnp.float32),
            pltpu.VMEM_SHARED((_PAIR,), jnp.float32),
            pltpu.VMEM((_CH, _LANES), jnp.int32),
            pltpu.VMEM((_CH, _LANES), jnp.float32),
            pltpu.VMEM((_CH, _LANES), jnp.float32),
            pltpu.SemaphoreType.DMA,
            pltpu.SemaphoreType.DMA,
        ],
    )(dest, vii, vjj)


def kernel(mask):
    tri = jnp.tril(jnp.ones((H, H), jnp.bfloat16))
    omega_i, idxm, kk, dest, vii, vjj = _dense_pass(tri, mask)
    II, JJ = _scatter_pass(dest, vii, vjj)
    scale = jnp.where(mask[0, 0, 0] < 2.0, jnp.float32(1), jnp.float32(0.5))
    Omega = jnp.transpose(omega_i, (0, 2, 3, 1)) * scale
    return Omega, idxm, II, JJ, kk.reshape(B, 2 * HW)


# speedup vs baseline: 1.2529x; 1.2529x over previous
import functools

import jax
import jax.numpy as jnp
from jax import lax
from jax.experimental import pallas as pl
from jax.experimental.pallas import tpu as pltpu
from jax.experimental.pallas import tpu_sc as plsc

B, H, W = 8, 512, 512
HW = H * W

_LANES = 128
_ROWS = B * HW // _LANES
_NW = 32
_RPW = _ROWS // _NW
_CH = 16
_NCH = _RPW // _CH


def _lane_cumsum_excl(x):
    total = x
    sh = 1
    while sh < W:
        z = jnp.zeros((1, sh), jnp.float32)
        total = total + jnp.concatenate([z, total[:, :-sh]], axis=1)
        sh *= 2
    return total - x


def _dense_body(tri_ref, mask_ref, omega_ref, idxm_ref, kk_ref, dest_ref,
                vii_ref, vjj_ref):
    b = pl.program_id(0)
    m = mask_ref[0]
    zrow = jnp.zeros((1, W), jnp.float32)
    zcol = jnp.zeros((H, 1), jnp.float32)
    m_down = jnp.concatenate([m[1:, :], zrow], axis=0)
    m_up = jnp.concatenate([zrow, m[:-1, :]], axis=0)
    m_right = jnp.concatenate([m[:, 1:], zcol], axis=1)
    m_left = jnp.concatenate([zcol, m[:, :-1]], axis=1)
    o0 = m * m_down
    o1 = m * m_up
    o2 = m * m_right
    o3 = m * m_left
    omega_ref[0, 0] = o0
    omega_ref[0, 1] = o1
    omega_ref[0, 2] = o2
    omega_ref[0, 3] = o3

    tri = tri_ref[...]

    c1 = jnp.dot(tri, m.astype(jnp.bfloat16),
                 preferred_element_type=jnp.float32)
    e1 = _lane_cumsum_excl(c1[H - 1:H, :])
    ranks = c1 + e1
    idxm = ranks * m
    idxm_ref[0] = idxm

    c2 = jnp.dot(tri, o2.astype(jnp.bfloat16),
                 preferred_element_type=jnp.float32)
    s2 = c2[H - 1:H, :]
    e2 = _lane_cumsum_excl(s2)
    r2 = c2 + e2
    counts = e2[:, W - 1:W] + s2[:, W - 1:W]

    vii_ref[0] = (idxm * o2).reshape(H * W // _LANES, _LANES)
    vjj_ref[0] = (jnp.concatenate([idxm[:, 1:], zcol], axis=1)
                  * o2).reshape(H * W // _LANES, _LANES)

    r2i = r2.astype(jnp.int32)
    counts_i = counts.astype(jnp.int32)
    hh = lax.broadcasted_iota(jnp.int32, (H, W), 0)
    ww = lax.broadcasted_iota(jnp.int32, (H, W), 1)
    j = ww * H + hh
    dest = jnp.where(o2 != 0, r2i - 1, counts_i + j - r2i)
    dest_ref[0] = (dest + lax.rem(b, 2) * HW).reshape(H * W // _LANES, _LANES)

    pos = lax.broadcasted_iota(jnp.int32, (1, 2 * HW), 1)
    sign = jnp.where(pos < HW, 1.0, -1.0)
    kk_ref[0] = jnp.where((pos & (HW - 1)) < counts_i, sign, 0.0)


@jax.jit
def _dense_pass(tri, mask):
    return pl.pallas_call(
        _dense_body,
        grid=(B,),
        in_specs=[
            pl.BlockSpec((H, H), lambda b: (0, 0)),
            pl.BlockSpec((1, H, W), lambda b: (b, 0, 0)),
        ],
        out_specs=[
            pl.BlockSpec((1, 4, H, W), lambda b: (b, 0, 0, 0)),
            pl.BlockSpec((1, H, W), lambda b: (b, 0, 0)),
            pl.BlockSpec((1, 1, 2 * HW), lambda b: (b, 0, 0)),
            pl.BlockSpec((1, HW // _LANES, _LANES), lambda b: (b, 0, 0)),
            pl.BlockSpec((1, HW // _LANES, _LANES), lambda b: (b, 0, 0)),
            pl.BlockSpec((1, HW // _LANES, _LANES), lambda b: (b, 0, 0)),
        ],
        out_shape=[
            jax.ShapeDtypeStruct((B, 4, H, W), jnp.float32),
            jax.ShapeDtypeStruct((B, H, W), jnp.float32),
            jax.ShapeDtypeStruct((B, 1, 2 * HW), jnp.float32),
            jax.ShapeDtypeStruct((B, HW // _LANES, _LANES), jnp.int32),
            jax.ShapeDtypeStruct((B, HW // _LANES, _LANES), jnp.float32),
            jax.ShapeDtypeStruct((B, HW // _LANES, _LANES), jnp.float32),
        ],
        compiler_params=pltpu.CompilerParams(
            dimension_semantics=("arbitrary",)),
    )(tri, mask)


_PAIR = 2 * HW
_RPT = 2048 * 2 // 16
_NCH2 = _RPT // _CH
_WR = _PAIR // 16


def _scatter_body(dest_hbm, vii_hbm, vjj_hbm, ii_out, jj_out,
                  sp_ii, sp_jj, idx_v0, a_v0, b_v0, idx_v1, a_v1, b_v1,
                  sem_ld, sem_a, sem_b):
    c = lax.axis_index("c")
    s = lax.axis_index("s")

    def one_round(r):
        b = 4 * c + 2 * r + s // 8
        lrow0 = (s % 8) * _RPT

        def start_loads(i, idx_v, a_v, b_v):
            rr = lrow0 + i * _CH
            pltpu.sync_copy(dest_hbm.at[b, pl.ds(rr, _CH)], idx_v)
            pltpu.sync_copy(vii_hbm.at[b, pl.ds(rr, _CH)], a_v)
            pltpu.sync_copy(vjj_hbm.at[b, pl.ds(rr, _CH)], b_v)

        def wait_loads(idx_v, a_v, b_v):
            pass

        def fire(idx_v, a_v, b_v):
            def body(j, carry2):
                pltpu.async_copy(a_v.at[j], sp_ii.at[idx_v.at[j]], sem_a)
                pltpu.async_copy(b_v.at[j], sp_jj.at[idx_v.at[j]], sem_b)
                return carry2

            lax.fori_loop(0, _CH, body, 0)

        def drain(idx_v, a_v, b_v):
            def body(j, carry2):
                pltpu.make_async_copy(a_v.at[j], sp_ii.at[idx_v.at[j]],
                                      sem_a).wait()
                pltpu.make_async_copy(b_v.at[j], sp_jj.at[idx_v.at[j]],
                                      sem_b).wait()
                return carry2

            lax.fori_loop(0, _CH, body, 0)

        start_loads(0, idx_v0, a_v0, b_v0)

        def pair(ip, carry):
            wait_loads(idx_v0, a_v0, b_v0)
            fire(idx_v0, a_v0, b_v0)
            start_loads(2 * ip + 1, idx_v1, a_v1, b_v1)
            drain(idx_v0, a_v0, b_v0)
            wait_loads(idx_v1, a_v1, b_v1)
            fire(idx_v1, a_v1, b_v1)

            @pl.when(ip < _NCH2 // 2 - 1)
            def _():
                start_loads(2 * ip + 2, idx_v0, a_v0, b_v0)

            drain(idx_v1, a_v1, b_v1)
            return carry

        lax.fori_loop(0, _NCH2 // 2, pair, 0)
        plsc.subcore_barrier()
        off = (s % 8) * _WR
        src_ii = sp_ii.at[pl.ds(s * _WR, _WR)]
        src_jj = sp_jj.at[pl.ds(s * _WR, _WR)]
        pltpu.sync_copy(src_ii, ii_out.at[b, pl.ds(off, _WR)])
        pltpu.sync_copy(src_ii, ii_out.at[b, pl.ds(HW + off, _WR)])
        pltpu.sync_copy(src_ii, jj_out.at[b, pl.ds(HW + off, _WR)])
        pltpu.sync_copy(src_jj, jj_out.at[b, pl.ds(off, _WR)])
        plsc.subcore_barrier()

    one_round(0)
    one_round(1)


@jax.jit
def _scatter_pass(dest, vii, vjj):
    return pl.kernel(
        _scatter_body,
        out_type=[
            jax.ShapeDtypeStruct((B, 2 * HW), jnp.float32),
            jax.ShapeDtypeStruct((B, 2 * HW), jnp.float32),
        ],
        mesh=plsc.VectorSubcoreMesh(core_axis_name="c", subcore_axis_name="s"),
        scratch_types=[
            pltpu.VMEM_SHARED((_PAIR,), jnp.float32),
            pltpu.VMEM_SHARED((_PAIR,), jnp.float32),
            pltpu.VMEM((_CH, _LANES), jnp.int32),
            pltpu.VMEM((_CH, _LANES), jnp.float32),
            pltpu.VMEM((_CH, _LANES), jnp.float32),
            pltpu.VMEM((_CH, _LANES), jnp.int32),
            pltpu.VMEM((_CH, _LANES), jnp.float32),
            pltpu.VMEM((_CH, _LANES), jnp.float32),
            pltpu.SemaphoreType.DMA,
            pltpu.SemaphoreType.DMA,
            pltpu.SemaphoreType.DMA,
        ],
    )(dest, vii, vjj)


def kernel(mask):
    tri = jnp.tril(jnp.ones((H, H), jnp.bfloat16))
    omega_i, idxm, kk, dest, vii, vjj = _dense_pass(tri, mask)
    II, JJ = _scatter_pass(dest, vii, vjj)
    Omega = jnp.transpose(omega_i, (0, 2, 3, 1))
    return Omega, idxm, II, JJ, kk.reshape(B, 2 * HW)


# speedup vs baseline: 1.4561x; 1.1622x over previous
import functools

import jax
import jax.numpy as jnp
from jax import lax
from jax.experimental import pallas as pl
from jax.experimental.pallas import tpu as pltpu
from jax.experimental.pallas import tpu_sc as plsc

B, H, W = 8, 512, 512
HW = H * W

_LANES = 128
_ROWS = B * HW // _LANES
_NW = 32
_RPW = _ROWS // _NW
_CH = 16
_NCH = _RPW // _CH


def _lane_cumsum_excl(x):
    total = x
    sh = 1
    while sh < W:
        z = jnp.zeros((1, sh), jnp.float32)
        total = total + jnp.concatenate([z, total[:, :-sh]], axis=1)
        sh *= 2
    return total - x


def _dense_body(tri_ref, mask_ref, omega_ref, idxm_ref, kk_ref, dest_ref,
                vii_ref, vjj_ref):
    b = pl.program_id(0)
    m = mask_ref[0]
    zrow = jnp.zeros((1, W), jnp.float32)
    zcol = jnp.zeros((H, 1), jnp.float32)
    m_down = jnp.concatenate([m[1:, :], zrow], axis=0)
    m_up = jnp.concatenate([zrow, m[:-1, :]], axis=0)
    m_right = jnp.concatenate([m[:, 1:], zcol], axis=1)
    m_left = jnp.concatenate([zcol, m[:, :-1]], axis=1)
    o0 = m * m_down
    o1 = m * m_up
    o2 = m * m_right
    o3 = m * m_left
    omega_ref[0, 0] = o0
    omega_ref[0, 1] = o1
    omega_ref[0, 2] = o2
    omega_ref[0, 3] = o3

    tri = tri_ref[...]

    c1 = jnp.dot(tri, m.astype(jnp.bfloat16),
                 preferred_element_type=jnp.float32)
    e1 = _lane_cumsum_excl(c1[H - 1:H, :])
    ranks = c1 + e1
    idxm = ranks * m
    idxm_ref[0] = idxm

    c2 = jnp.dot(tri, o2.astype(jnp.bfloat16),
                 preferred_element_type=jnp.float32)
    s2 = c2[H - 1:H, :]
    e2 = _lane_cumsum_excl(s2)
    r2 = c2 + e2
    counts = e2[:, W - 1:W] + s2[:, W - 1:W]

    vii_ref[0] = (idxm * o2).reshape(H * W // _LANES, _LANES)
    vjj_ref[0] = (jnp.concatenate([idxm[:, 1:], zcol], axis=1)
                  * o2).reshape(H * W // _LANES, _LANES)

    r2i = r2.astype(jnp.int32)
    counts_i = counts.astype(jnp.int32)
    hh = lax.broadcasted_iota(jnp.int32, (H, W), 0)
    ww = lax.broadcasted_iota(jnp.int32, (H, W), 1)
    j = ww * H + hh
    dest = jnp.where(o2 != 0, r2i - 1, counts_i + j - r2i)
    dest_ref[0] = (dest + lax.rem(b, 2) * HW).reshape(H * W // _LANES, _LANES)

    pos = lax.broadcasted_iota(jnp.int32, (1, 2 * HW), 1)
    sign = jnp.where(pos < HW, 1.0, -1.0)
    kk_ref[0] = jnp.where((pos & (HW - 1)) < counts_i, sign, 0.0)


@jax.jit
def _dense_pass(tri, mask):
    return pl.pallas_call(
        _dense_body,
        grid=(B,),
        in_specs=[
            pl.BlockSpec((H, H), lambda b: (0, 0)),
            pl.BlockSpec((1, H, W), lambda b: (b, 0, 0)),
        ],
        out_specs=[
            pl.BlockSpec((1, 4, H, W), lambda b: (b, 0, 0, 0)),
            pl.BlockSpec((1, H, W), lambda b: (b, 0, 0)),
            pl.BlockSpec((1, 1, 2 * HW), lambda b: (b, 0, 0)),
            pl.BlockSpec((1, HW // _LANES, _LANES), lambda b: (b, 0, 0)),
            pl.BlockSpec((1, HW // _LANES, _LANES), lambda b: (b, 0, 0)),
            pl.BlockSpec((1, HW // _LANES, _LANES), lambda b: (b, 0, 0)),
        ],
        out_shape=[
            jax.ShapeDtypeStruct((B, 4, H, W), jnp.float32),
            jax.ShapeDtypeStruct((B, H, W), jnp.float32),
            jax.ShapeDtypeStruct((B, 1, 2 * HW), jnp.float32),
            jax.ShapeDtypeStruct((B, HW // _LANES, _LANES), jnp.int32),
            jax.ShapeDtypeStruct((B, HW // _LANES, _LANES), jnp.float32),
            jax.ShapeDtypeStruct((B, HW // _LANES, _LANES), jnp.float32),
        ],
        compiler_params=pltpu.CompilerParams(
            dimension_semantics=("arbitrary",)),
    )(tri, mask)


_PAIR = 2 * HW
_RPT = 2048 * 2 // 16
_NCH2 = _RPT // _CH
_WR = _PAIR // 16


def _scatter_body(dest_hbm, vii_hbm, vjj_hbm, ii_out, jj_out,
                  sp_ii, sp_jj, idx_v0, a_v0, b_v0, idx_v1, a_v1, b_v1,
                  sem_ld, sem_a, sem_b):
    c = lax.axis_index("c")
    s = lax.axis_index("s")

    def one_round(r):
        b = 4 * c + 2 * r + s // 8
        lrow0 = (s % 8) * _RPT

        def start_loads(i, idx_v, a_v, b_v):
            rr = lrow0 + i * _CH
            pltpu.async_copy(dest_hbm.at[b, pl.ds(rr, _CH)], idx_v, sem_ld)
            pltpu.async_copy(vii_hbm.at[b, pl.ds(rr, _CH)], a_v, sem_ld)
            pltpu.async_copy(vjj_hbm.at[b, pl.ds(rr, _CH)], b_v, sem_ld)

        def wait_loads(i, idx_v, a_v, b_v):
            rr = lrow0 + i * _CH
            pltpu.make_async_copy(dest_hbm.at[b, pl.ds(rr, _CH)],
                                  idx_v, sem_ld).wait()
            pltpu.make_async_copy(vii_hbm.at[b, pl.ds(rr, _CH)],
                                  a_v, sem_ld).wait()
            pltpu.make_async_copy(vjj_hbm.at[b, pl.ds(rr, _CH)],
                                  b_v, sem_ld).wait()

        def fire(idx_v, a_v, b_v):
            def body(j, carry2):
                pltpu.async_copy(a_v.at[j], sp_ii.at[idx_v.at[j]], sem_a)
                pltpu.async_copy(b_v.at[j], sp_jj.at[idx_v.at[j]], sem_b)
                return carry2

            lax.fori_loop(0, _CH, body, 0)

        def drain(idx_v, a_v, b_v):
            def body(j, carry2):
                pltpu.make_async_copy(a_v.at[j], sp_ii.at[idx_v.at[j]],
                                      sem_a).wait()
                pltpu.make_async_copy(b_v.at[j], sp_jj.at[idx_v.at[j]],
                                      sem_b).wait()
                return carry2

            lax.fori_loop(0, _CH, body, 0)

        start_loads(0, idx_v0, a_v0, b_v0)

        def pair(ip, carry):
            wait_loads(2 * ip, idx_v0, a_v0, b_v0)
            fire(idx_v0, a_v0, b_v0)
            start_loads(2 * ip + 1, idx_v1, a_v1, b_v1)
            drain(idx_v0, a_v0, b_v0)
            wait_loads(2 * ip + 1, idx_v1, a_v1, b_v1)
            fire(idx_v1, a_v1, b_v1)

            @pl.when(ip < _NCH2 // 2 - 1)
            def _():
                start_loads(2 * ip + 2, idx_v0, a_v0, b_v0)

            drain(idx_v1, a_v1, b_v1)
            return carry

        lax.fori_loop(0, _NCH2 // 2, pair, 0)
        plsc.subcore_barrier()
        off = (s % 8) * _WR
        src_ii = sp_ii.at[pl.ds(s * _WR, _WR)]
        src_jj = sp_jj.at[pl.ds(s * _WR, _WR)]
        pltpu.sync_copy(src_ii, ii_out.at[b, pl.ds(off, _WR)])
        pltpu.sync_copy(src_ii, ii_out.at[b, pl.ds(HW + off, _WR)])
        pltpu.sync_copy(src_ii, jj_out.at[b, pl.ds(HW + off, _WR)])
        pltpu.sync_copy(src_jj, jj_out.at[b, pl.ds(off, _WR)])
        plsc.subcore_barrier()

    one_round(0)
    one_round(1)


@jax.jit
def _scatter_pass(dest, vii, vjj):
    return pl.kernel(
        _scatter_body,
        out_type=[
            jax.ShapeDtypeStruct((B, 2 * HW), jnp.float32),
            jax.ShapeDtypeStruct((B, 2 * HW), jnp.float32),
        ],
        mesh=plsc.VectorSubcoreMesh(core_axis_name="c", subcore_axis_name="s"),
        scratch_types=[
            pltpu.VMEM_SHARED((_PAIR,), jnp.float32),
            pltpu.VMEM_SHARED((_PAIR,), jnp.float32),
            pltpu.VMEM((_CH, _LANES), jnp.int32),
            pltpu.VMEM((_CH, _LANES), jnp.float32),
            pltpu.VMEM((_CH, _LANES), jnp.float32),
            pltpu.VMEM((_CH, _LANES), jnp.int32),
            pltpu.VMEM((_CH, _LANES), jnp.float32),
            pltpu.VMEM((_CH, _LANES), jnp.float32),
            pltpu.SemaphoreType.DMA,
            pltpu.SemaphoreType.DMA,
            pltpu.SemaphoreType.DMA,
        ],
    )(dest, vii, vjj)


def kernel(mask):
    tri = jnp.tril(jnp.ones((H, H), jnp.bfloat16))
    omega_i, idxm, kk, dest, vii, vjj = _dense_pass(tri, mask)
    II, JJ = _scatter_pass(dest, vii, vjj)
    Omega = jnp.transpose(omega_i, (0, 2, 3, 1))
    return Omega, idxm, II, JJ, kk.reshape(B, 2 * HW)


# speedup vs baseline: 1.4789x; 1.0157x over previous
import functools

import jax
import jax.numpy as jnp
from jax import lax
from jax.experimental import pallas as pl
from jax.experimental.pallas import tpu as pltpu
from jax.experimental.pallas import tpu_sc as plsc

B, H, W = 8, 512, 512
HW = H * W

_LANES = 128
_ROWS = B * HW // _LANES
_NW = 32
_RPW = _ROWS // _NW
_CH = 32
_NCH = _RPW // _CH


def _lane_cumsum_excl(x):
    total = x
    sh = 1
    while sh < W:
        z = jnp.zeros((1, sh), jnp.float32)
        total = total + jnp.concatenate([z, total[:, :-sh]], axis=1)
        sh *= 2
    return total - x


def _dense_body(tri_ref, mask_ref, omega_ref, idxm_ref, kk_ref, dest_ref,
                vii_ref, vjj_ref):
    b = pl.program_id(0)
    m = mask_ref[0]
    zrow = jnp.zeros((1, W), jnp.float32)
    zcol = jnp.zeros((H, 1), jnp.float32)
    m_down = jnp.concatenate([m[1:, :], zrow], axis=0)
    m_up = jnp.concatenate([zrow, m[:-1, :]], axis=0)
    m_right = jnp.concatenate([m[:, 1:], zcol], axis=1)
    m_left = jnp.concatenate([zcol, m[:, :-1]], axis=1)
    o0 = m * m_down
    o1 = m * m_up
    o2 = m * m_right
    o3 = m * m_left
    omega_ref[0, 0] = o0
    omega_ref[0, 1] = o1
    omega_ref[0, 2] = o2
    omega_ref[0, 3] = o3

    tri = tri_ref[...]

    c1 = jnp.dot(tri, m.astype(jnp.bfloat16),
                 preferred_element_type=jnp.float32)
    e1 = _lane_cumsum_excl(c1[H - 1:H, :])
    ranks = c1 + e1
    idxm = ranks * m
    idxm_ref[0] = idxm

    c2 = jnp.dot(tri, o2.astype(jnp.bfloat16),
                 preferred_element_type=jnp.float32)
    s2 = c2[H - 1:H, :]
    e2 = _lane_cumsum_excl(s2)
    r2 = c2 + e2
    counts = e2[:, W - 1:W] + s2[:, W - 1:W]

    vii_ref[0] = (idxm * o2).reshape(H * W // _LANES, _LANES)
    vjj_ref[0] = (jnp.concatenate([idxm[:, 1:], zcol], axis=1)
                  * o2).reshape(H * W // _LANES, _LANES)

    r2i = r2.astype(jnp.int32)
    counts_i = counts.astype(jnp.int32)
    hh = lax.broadcasted_iota(jnp.int32, (H, W), 0)
    ww = lax.broadcasted_iota(jnp.int32, (H, W), 1)
    j = ww * H + hh
    dest = jnp.where(o2 != 0, r2i - 1, counts_i + j - r2i)
    dest_ref[0] = (dest + lax.rem(b, 2) * HW).reshape(H * W // _LANES, _LANES)

    pos = lax.broadcasted_iota(jnp.int32, (1, 2 * HW), 1)
    sign = jnp.where(pos < HW, 1.0, -1.0)
    kk_ref[0] = jnp.where((pos & (HW - 1)) < counts_i, sign, 0.0)


@jax.jit
def _dense_pass(tri, mask):
    return pl.pallas_call(
        _dense_body,
        grid=(B,),
        in_specs=[
            pl.BlockSpec((H, H), lambda b: (0, 0)),
            pl.BlockSpec((1, H, W), lambda b: (b, 0, 0)),
        ],
        out_specs=[
            pl.BlockSpec((1, 4, H, W), lambda b: (b, 0, 0, 0)),
            pl.BlockSpec((1, H, W), lambda b: (b, 0, 0)),
            pl.BlockSpec((1, 1, 2 * HW), lambda b: (b, 0, 0)),
            pl.BlockSpec((1, HW // _LANES, _LANES), lambda b: (b, 0, 0)),
            pl.BlockSpec((1, HW // _LANES, _LANES), lambda b: (b, 0, 0)),
            pl.BlockSpec((1, HW // _LANES, _LANES), lambda b: (b, 0, 0)),
        ],
        out_shape=[
            jax.ShapeDtypeStruct((B, 4, H, W), jnp.float32),
            jax.ShapeDtypeStruct((B, H, W), jnp.float32),
            jax.ShapeDtypeStruct((B, 1, 2 * HW), jnp.float32),
            jax.ShapeDtypeStruct((B, HW // _LANES, _LANES), jnp.int32),
            jax.ShapeDtypeStruct((B, HW // _LANES, _LANES), jnp.float32),
            jax.ShapeDtypeStruct((B, HW // _LANES, _LANES), jnp.float32),
        ],
        compiler_params=pltpu.CompilerParams(
            dimension_semantics=("arbitrary",)),
    )(tri, mask)


_PAIR = 2 * HW
_RPT = 2048 * 2 // 16
_NCH2 = _RPT // _CH
_WR = _PAIR // 16


def _scatter_body(dest_hbm, vii_hbm, vjj_hbm, ii_out, jj_out,
                  sp_ii, sp_jj, idx_v0, a_v0, b_v0, idx_v1, a_v1, b_v1,
                  sem_ld, sem_a, sem_b):
    c = lax.axis_index("c")
    s = lax.axis_index("s")

    def one_round(r):
        b = 4 * c + 2 * r + s // 8
        lrow0 = (s % 8) * _RPT

        def start_loads(i, idx_v, a_v, b_v):
            rr = lrow0 + i * _CH
            pltpu.async_copy(dest_hbm.at[b, pl.ds(rr, _CH)], idx_v, sem_ld)
            pltpu.async_copy(vii_hbm.at[b, pl.ds(rr, _CH)], a_v, sem_ld)
            pltpu.async_copy(vjj_hbm.at[b, pl.ds(rr, _CH)], b_v, sem_ld)

        def wait_loads(i, idx_v, a_v, b_v):
            rr = lrow0 + i * _CH
            pltpu.make_async_copy(dest_hbm.at[b, pl.ds(rr, _CH)],
                                  idx_v, sem_ld).wait()
            pltpu.make_async_copy(vii_hbm.at[b, pl.ds(rr, _CH)],
                                  a_v, sem_ld).wait()
            pltpu.make_async_copy(vjj_hbm.at[b, pl.ds(rr, _CH)],
                                  b_v, sem_ld).wait()

        def fire(idx_v, a_v, b_v):
            def body(j, carry2):
                pltpu.async_copy(a_v.at[j], sp_ii.at[idx_v.at[j]], sem_a)
                pltpu.async_copy(b_v.at[j], sp_jj.at[idx_v.at[j]], sem_b)
                return carry2

            lax.fori_loop(0, _CH, body, 0)

        def drain(idx_v, a_v, b_v):
            def body(j, carry2):
                pltpu.make_async_copy(a_v.at[j], sp_ii.at[idx_v.at[j]],
                                      sem_a).wait()
                pltpu.make_async_copy(b_v.at[j], sp_jj.at[idx_v.at[j]],
                                      sem_b).wait()
                return carry2

            lax.fori_loop(0, _CH, body, 0)

        start_loads(0, idx_v0, a_v0, b_v0)

        def pair(ip, carry):
            wait_loads(2 * ip, idx_v0, a_v0, b_v0)
            fire(idx_v0, a_v0, b_v0)
            start_loads(2 * ip + 1, idx_v1, a_v1, b_v1)
            drain(idx_v0, a_v0, b_v0)
            wait_loads(2 * ip + 1, idx_v1, a_v1, b_v1)
            fire(idx_v1, a_v1, b_v1)

            @pl.when(ip < _NCH2 // 2 - 1)
            def _():
                start_loads(2 * ip + 2, idx_v0, a_v0, b_v0)

            drain(idx_v1, a_v1, b_v1)
            return carry

        lax.fori_loop(0, _NCH2 // 2, pair, 0)
        plsc.subcore_barrier()
        off = (s % 8) * _WR
        src_ii = sp_ii.at[pl.ds(s * _WR, _WR)]
        src_jj = sp_jj.at[pl.ds(s * _WR, _WR)]
        pltpu.sync_copy(src_ii, ii_out.at[b, pl.ds(off, _WR)])
        pltpu.sync_copy(src_ii, ii_out.at[b, pl.ds(HW + off, _WR)])
        pltpu.sync_copy(src_ii, jj_out.at[b, pl.ds(HW + off, _WR)])
        pltpu.sync_copy(src_jj, jj_out.at[b, pl.ds(off, _WR)])
        plsc.subcore_barrier()

    one_round(0)
    one_round(1)


@jax.jit
def _scatter_pass(dest, vii, vjj):
    return pl.kernel(
        _scatter_body,
        out_type=[
            jax.ShapeDtypeStruct((B, 2 * HW), jnp.float32),
            jax.ShapeDtypeStruct((B, 2 * HW), jnp.float32),
        ],
        mesh=plsc.VectorSubcoreMesh(core_axis_name="c", subcore_axis_name="s"),
        scratch_types=[
            pltpu.VMEM_SHARED((_PAIR,), jnp.float32),
            pltpu.VMEM_SHARED((_PAIR,), jnp.float32),
            pltpu.VMEM((_CH, _LANES), jnp.int32),
            pltpu.VMEM((_CH, _LANES), jnp.float32),
            pltpu.VMEM((_CH, _LANES), jnp.float32),
            pltpu.VMEM((_CH, _LANES), jnp.int32),
            pltpu.VMEM((_CH, _LANES), jnp.float32),
            pltpu.VMEM((_CH, _LANES), jnp.float32),
            pltpu.SemaphoreType.DMA,
            pltpu.SemaphoreType.DMA,
            pltpu.SemaphoreType.DMA,
        ],
    )(dest, vii, vjj)


def kernel(mask):
    tri = jnp.tril(jnp.ones((H, H), jnp.bfloat16))
    omega_i, idxm, kk, dest, vii, vjj = _dense_pass(tri, mask)
    II, JJ = _scatter_pass(dest, vii, vjj)
    Omega = jnp.transpose(omega_i, (0, 2, 3, 1))
    return Omega, idxm, II, JJ, kk.reshape(B, 2 * HW)


# speedup vs baseline: 1.5227x; 1.0296x over previous
import functools

import jax
import jax.numpy as jnp
from jax import lax
from jax.experimental import pallas as pl
from jax.experimental.pallas import tpu as pltpu
from jax.experimental.pallas import tpu_sc as plsc

B, H, W = 8, 512, 512
HW = H * W

_LANES = 128
_ROWS = B * HW // _LANES
_NW = 32
_RPW = _ROWS // _NW
_CH = 32
_NCH = _RPW // _CH


def _lane_cumsum_excl(x):
    total = x
    sh = 1
    while sh < W:
        z = jnp.zeros((1, sh), jnp.float32)
        total = total + jnp.concatenate([z, total[:, :-sh]], axis=1)
        sh *= 2
    return total - x


def _shifts(m):
    zrow = jnp.zeros((1, W), jnp.float32)
    zcol = jnp.zeros((H, 1), jnp.float32)
    m_down = jnp.concatenate([m[1:, :], zrow], axis=0)
    m_up = jnp.concatenate([zrow, m[:-1, :]], axis=0)
    m_right = jnp.concatenate([m[:, 1:], zcol], axis=1)
    m_left = jnp.concatenate([zcol, m[:, :-1]], axis=1)
    return m_down, m_up, m_right, m_left


def _cm_rank(tri, x):
    c = jnp.dot(tri, x.astype(jnp.bfloat16), preferred_element_type=jnp.float32)
    s = c[H - 1:H, :]
    e = _lane_cumsum_excl(s)
    return c + e, e[:, W - 1:W] + s[:, W - 1:W]


def _sc_feed_body(tri_ref, mask_ref, dest_ref, vii_ref, vjj_ref):
    b = pl.program_id(0)
    m = mask_ref[0]
    zcol = jnp.zeros((H, 1), jnp.float32)
    m_right = jnp.concatenate([m[:, 1:], zcol], axis=1)
    o2 = m * m_right
    tri = tri_ref[...]

    ranks, _ = _cm_rank(tri, m)
    idxm = ranks * m
    r2, counts = _cm_rank(tri, o2)

    vii_ref[0] = (idxm * o2).reshape(H * W // _LANES, _LANES)
    vjj_ref[0] = (jnp.concatenate([idxm[:, 1:], zcol], axis=1)
                  * o2).reshape(H * W // _LANES, _LANES)

    r2i = r2.astype(jnp.int32)
    counts_i = counts.astype(jnp.int32)
    hh = lax.broadcasted_iota(jnp.int32, (H, W), 0)
    ww = lax.broadcasted_iota(jnp.int32, (H, W), 1)
    j = ww * H + hh
    dest = jnp.where(o2 != 0, r2i - 1, counts_i + j - r2i)
    dest_ref[0] = (dest + lax.rem(b, 2) * HW).reshape(H * W // _LANES, _LANES)


def _dense_body(tri_ref, mask_ref, omega_ref, idxm_ref, kk_ref):
    m = mask_ref[0]
    m_down, m_up, m_right, m_left = _shifts(m)
    o2 = m * m_right
    omega_ref[0, 0] = m * m_down
    omega_ref[0, 1] = m * m_up
    omega_ref[0, 2] = o2
    omega_ref[0, 3] = m * m_left

    tri = tri_ref[...]
    ranks, _ = _cm_rank(tri, m)
    idxm_ref[0] = ranks * m
    _, counts = _cm_rank(tri, o2)
    counts_i = counts.astype(jnp.int32)

    pos = lax.broadcasted_iota(jnp.int32, (1, 2 * HW), 1)
    sign = jnp.where(pos < HW, 1.0, -1.0)
    kk_ref[0] = jnp.where((pos & (HW - 1)) < counts_i, sign, 0.0)


@jax.jit
def _sc_feed_pass(tri, mask):
    return pl.pallas_call(
        _sc_feed_body,
        grid=(B,),
        in_specs=[
            pl.BlockSpec((H, H), lambda b: (0, 0)),
            pl.BlockSpec((1, H, W), lambda b: (b, 0, 0)),
        ],
        out_specs=[
            pl.BlockSpec((1, HW // _LANES, _LANES), lambda b: (b, 0, 0)),
            pl.BlockSpec((1, HW // _LANES, _LANES), lambda b: (b, 0, 0)),
            pl.BlockSpec((1, HW // _LANES, _LANES), lambda b: (b, 0, 0)),
        ],
        out_shape=[
            jax.ShapeDtypeStruct((B, HW // _LANES, _LANES), jnp.int32),
            jax.ShapeDtypeStruct((B, HW // _LANES, _LANES), jnp.float32),
            jax.ShapeDtypeStruct((B, HW // _LANES, _LANES), jnp.float32),
        ],
        compiler_params=pltpu.CompilerParams(
            dimension_semantics=("arbitrary",)),
    )(tri, mask)


@jax.jit
def _dense_pass(tri, mask):
    return pl.pallas_call(
        _dense_body,
        grid=(B,),
        in_specs=[
            pl.BlockSpec((H, H), lambda b: (0, 0)),
            pl.BlockSpec((1, H, W), lambda b: (b, 0, 0)),
        ],
        out_specs=[
            pl.BlockSpec((1, 4, H, W), lambda b: (b, 0, 0, 0)),
            pl.BlockSpec((1, H, W), lambda b: (b, 0, 0)),
            pl.BlockSpec((1, 1, 2 * HW), lambda b: (b, 0, 0)),
        ],
        out_shape=[
            jax.ShapeDtypeStruct((B, 4, H, W), jnp.float32),
            jax.ShapeDtypeStruct((B, H, W), jnp.float32),
            jax.ShapeDtypeStruct((B, 1, 2 * HW), jnp.float32),
        ],
        compiler_params=pltpu.CompilerParams(
            dimension_semantics=("arbitrary",)),
    )(tri, mask)


_PAIR = 2 * HW
_RPT = 2048 * 2 // 16
_NCH2 = _RPT // _CH
_WR = _PAIR // 16


def _scatter_body(dest_hbm, vii_hbm, vjj_hbm, ii_out, jj_out,
                  sp_ii, sp_jj, idx_v0, a_v0, b_v0, idx_v1, a_v1, b_v1,
                  sem_ld, sem_a, sem_b):
    c = lax.axis_index("c")
    s = lax.axis_index("s")

    def one_round(r):
        b = 4 * c + 2 * r + s // 8
        lrow0 = (s % 8) * _RPT

        def start_loads(i, idx_v, a_v, b_v):
            rr = lrow0 + i * _CH
            pltpu.async_copy(dest_hbm.at[b, pl.ds(rr, _CH)], idx_v, sem_ld)
            pltpu.async_copy(vii_hbm.at[b, pl.ds(rr, _CH)], a_v, sem_ld)
            pltpu.async_copy(vjj_hbm.at[b, pl.ds(rr, _CH)], b_v, sem_ld)

        def wait_loads(i, idx_v, a_v, b_v):
            rr = lrow0 + i * _CH
            pltpu.make_async_copy(dest_hbm.at[b, pl.ds(rr, _CH)],
                                  idx_v, sem_ld).wait()
            pltpu.make_async_copy(vii_hbm.at[b, pl.ds(rr, _CH)],
                                  a_v, sem_ld).wait()
            pltpu.make_async_copy(vjj_hbm.at[b, pl.ds(rr, _CH)],
                                  b_v, sem_ld).wait()

        def fire(idx_v, a_v, b_v):
            def body(j, carry2):
                pltpu.async_copy(a_v.at[j], sp_ii.at[idx_v.at[j]], sem_a)
                pltpu.async_copy(b_v.at[j], sp_jj.at[idx_v.at[j]], sem_b)
                return carry2

            lax.fori_loop(0, _CH, body, 0)

        def drain(idx_v, a_v, b_v):
            def body(j, carry2):
                pltpu.make_async_copy(a_v.at[j], sp_ii.at[idx_v.at[j]],
                                      sem_a).wait()
                pltpu.make_async_copy(b_v.at[j], sp_jj.at[idx_v.at[j]],
                                      sem_b).wait()
                return carry2

            lax.fori_loop(0, _CH, body, 0)

        start_loads(0, idx_v0, a_v0, b_v0)

        def pair(ip, carry):
            wait_loads(2 * ip, idx_v0, a_v0, b_v0)
            fire(idx_v0, a_v0, b_v0)
            start_loads(2 * ip + 1, idx_v1, a_v1, b_v1)
            drain(idx_v0, a_v0, b_v0)
            wait_loads(2 * ip + 1, idx_v1, a_v1, b_v1)
            fire(idx_v1, a_v1, b_v1)

            @pl.when(ip < _NCH2 // 2 - 1)
            def _():
                start_loads(2 * ip + 2, idx_v0, a_v0, b_v0)

            drain(idx_v1, a_v1, b_v1)
            return carry

        lax.fori_loop(0, _NCH2 // 2, pair, 0)
        plsc.subcore_barrier()
        off = (s % 8) * _WR
        src_ii = sp_ii.at[pl.ds(s * _WR, _WR)]
        src_jj = sp_jj.at[pl.ds(s * _WR, _WR)]
        pltpu.sync_copy(src_ii, ii_out.at[b, pl.ds(off, _WR)])
        pltpu.sync_copy(src_ii, ii_out.at[b, pl.ds(HW + off, _WR)])
        pltpu.sync_copy(src_ii, jj_out.at[b, pl.ds(HW + off, _WR)])
        pltpu.sync_copy(src_jj, jj_out.at[b, pl.ds(off, _WR)])
        plsc.subcore_barrier()

    one_round(0)
    one_round(1)


@jax.jit
def _scatter_pass(dest, vii, vjj):
    return pl.kernel(
        _scatter_body,
        out_type=[
            jax.ShapeDtypeStruct((B, 2 * HW), jnp.float32),
            jax.ShapeDtypeStruct((B, 2 * HW), jnp.float32),
        ],
        mesh=plsc.VectorSubcoreMesh(core_axis_name="c", subcore_axis_name="s"),
        scratch_types=[
            pltpu.VMEM_SHARED((_PAIR,), jnp.float32),
            pltpu.VMEM_SHARED((_PAIR,), jnp.float32),
            pltpu.VMEM((_CH, _LANES), jnp.int32),
            pltpu.VMEM((_CH, _LANES), jnp.float32),
            pltpu.VMEM((_CH, _LANES), jnp.float32),
            pltpu.VMEM((_CH, _LANES), jnp.int32),
            pltpu.VMEM((_CH, _LANES), jnp.float32),
            pltpu.VMEM((_CH, _LANES), jnp.float32),
            pltpu.SemaphoreType.DMA,
            pltpu.SemaphoreType.DMA,
            pltpu.SemaphoreType.DMA,
        ],
    )(dest, vii, vjj)


def kernel(mask):
    tri = jnp.tril(jnp.ones((H, H), jnp.bfloat16))
    dest, vii, vjj = _sc_feed_pass(tri, mask)
    II, JJ = _scatter_pass(dest, vii, vjj)
    omega_i, idxm, kk = _dense_pass(tri, mask)
    Omega = jnp.transpose(omega_i, (0, 2, 3, 1))
    return Omega, idxm, II, JJ, kk.reshape(B, 2 * HW)
